# baseline (speedup 1.0000x reference)
import functools

import jax
import jax.numpy as jnp
from jax import lax
from jax.experimental import pallas as pl
from jax.experimental.pallas import tpu as pltpu
from jax.experimental.pallas import tpu_sc as plsc

D_MODEL = 64
SCALE = 8.0
NB = 4
CHUNK = 256


def _body(n_chunks, b_per_w, idx_hbm, table_hbm, out_hbm, idx_v, rows_v,
          g0, g1, g2, g3, o0, o1, o2, o3):
  gsems = (g0, g1, g2, g3)
  osems = (o0, o1, o2, o3)
  wid = lax.axis_index("s") * 2 + lax.axis_index("c")
  base = wid * b_per_w

  pltpu.sync_copy(idx_hbm.at[pl.ds(base, b_per_w)], idx_v)

  def start_gather(c, b):
    off = pl.multiple_of(c * CHUNK, CHUNK)
    pltpu.async_copy(
        table_hbm.at[idx_v.at[pl.ds(off, CHUNK)]], rows_v.at[b], gsems[b])

  def wait_gather(b):
    pltpu.make_async_copy(
        table_hbm.at[idx_v.at[pl.ds(0, CHUNK)]], rows_v.at[b],
        gsems[b]).wait()

  def start_out(c, b):
    row0 = base + pl.multiple_of(c * CHUNK, CHUNK)
    pltpu.async_copy(rows_v.at[b], out_hbm.at[pl.ds(row0, CHUNK)], osems[b])

  def wait_out(b):
    pltpu.make_async_copy(
        rows_v.at[b], out_hbm.at[pl.ds(base, CHUNK)], osems[b]).wait()

  def scale_slot(b):
    def mul_row(r, carry):
      for j in range(D_MODEL // 16):
        sl = (b, r, pl.ds(j * 16, 16))
        rows_v[sl] = rows_v[sl] * SCALE
      return carry
    lax.fori_loop(0, CHUNK, mul_row, 0, unroll=4)

  for b in range(NB):
    start_gather(b, b)

  def outer(i, carry):
    for b in range(NB):
      c = i * NB + b
      wait_gather(b)
      scale_slot(b)
      start_out(c, b)
      wait_out(b)
      start_gather(c + NB, b)
    return carry
  lax.fori_loop(0, n_chunks // NB - 1, outer, 0)

  for b in range(NB):
    c = n_chunks - NB + b
    wait_gather(b)
    scale_slot(b)
    start_out(c, b)
  for b in range(NB):
    wait_out(b)


@functools.partial(jax.jit, static_argnames=())
def kernel(x, table):
  b0, b1 = x.shape
  n = b0 * b1
  vocab, d = table.shape
  assert d == D_MODEL
  nw = 32
  b_per_w = n // nw
  assert b_per_w * nw == n and b_per_w % (NB * CHUNK) == 0
  n_chunks = b_per_w // CHUNK

  xf = x.reshape(n).astype(jnp.int32)
  mesh = plsc.VectorSubcoreMesh(core_axis_name="c", subcore_axis_name="s")
  run = pl.kernel(
      functools.partial(_body, n_chunks, b_per_w),
      mesh=mesh,
      out_type=jax.ShapeDtypeStruct((n, D_MODEL), jnp.float32),
      scratch_types=[
          pltpu.VMEM((b_per_w,), jnp.int32),
          pltpu.VMEM((NB, CHUNK, D_MODEL), jnp.float32),
      ] + [pltpu.SemaphoreType.DMA] * (2 * NB),
      compiler_params=pltpu.CompilerParams(use_tc_tiling_on_sc=False),
  )
  out = run(xf, table)
  return out.reshape(b0, b1, D_MODEL)
